# baseline (speedup 1.0000x reference)
import functools

import jax
import jax.numpy as jnp
from jax import lax
from jax.experimental import pallas as pl
from jax.experimental.pallas import tpu as pltpu
from jax.experimental.pallas import tpu_sc as plsc

V, D, B, N = 100000, 64, 16384, 20
NC, NS, L = 2, 16, 16
NW = NC * NS
BPW = B // NW
G = 64
NG = BPW // G
SG = G // L
IDX_CHUNK = 128
PR = BPW // 128
NR = BPW * N // 128


def _sc_scores(W_target, W_context, target_ids, context_ids, neg_ids):
    mesh = plsc.VectorSubcoreMesh(core_axis_name="c", subcore_axis_name="s")

    @functools.partial(
        pl.kernel,
        out_type=(
            jax.ShapeDtypeStruct((B // 128, 128), jnp.float32),
            jax.ShapeDtypeStruct((B * N // 128, 128), jnp.float32),
        ),
        mesh=mesh,
        scratch_types=[
            pltpu.VMEM((BPW,), jnp.int32),
            pltpu.VMEM((BPW,), jnp.int32),
            pltpu.VMEM((G, 128), jnp.int32),
            pltpu.VMEM((G * N,), jnp.int32),
            pltpu.VMEM((G, D), jnp.float32),
            pltpu.VMEM((G, D), jnp.float32),
            pltpu.VMEM((G * N, D), jnp.float32),
            pltpu.VMEM((PR, 128), jnp.float32),
            pltpu.VMEM((NR, 128), jnp.float32),
            pltpu.SemaphoreType.DMA,
        ],
        compiler_params=pltpu.CompilerParams(needs_layout_passes=False,
                                             use_tc_tiling_on_sc=False),
    )
    def score_kernel(wt_hbm, wc_hbm, tid_hbm, cid_hbm, nid_hbm,
                     pos_hbm, neg_hbm,
                     idx_t, idx_c, idx_n2, idx_n, t_rows, c_rows, n_rows,
                     pos_v, neg_v, sem):
        wid = lax.axis_index("s") * NC + lax.axis_index("c")
        base = wid * BPW

        pltpu.sync_copy(tid_hbm.at[pl.ds(base, BPW)], idx_t)
        pltpu.sync_copy(cid_hbm.at[pl.ds(base, BPW)], idx_c)

        lane = lax.iota(jnp.int32, L)

        for g in range(NG):
            pltpu.sync_copy(nid_hbm.at[pl.ds(base + g * G, G), :], idx_n2)

            def repack(k, _):
                for u in range(4):
                    j = (k * 4 + u) * L + lane
                    r = j // N
                    c = j - r * N
                    idx_n[pl.ds((k * 4 + u) * L, L)] = plsc.load_gather(
                        idx_n2, [r, c])
                return 0

            lax.fori_loop(0, G * N // (4 * L), repack, 0)

            copies = [
                pltpu.async_copy(wt_hbm.at[idx_t.at[pl.ds(g * G, G)]],
                                 t_rows, sem),
                pltpu.async_copy(wc_hbm.at[idx_c.at[pl.ds(g * G, G)]],
                                 c_rows, sem),
            ]
            for j in range(G * N // IDX_CHUNK):
                copies.append(pltpu.async_copy(
                    wc_hbm.at[idx_n.at[pl.ds(j * IDX_CHUNK, IDX_CHUNK)]],
                    n_rows.at[pl.ds(j * IDX_CHUNK, IDX_CHUNK)], sem))
            for cp in copies:
                cp.wait()

            for sg in range(SG):
                rows_tc = sg * L + lane
                rows_nb = rows_tc * N

                def body(d, carry, rows_tc=rows_tc, rows_nb=rows_nb):
                    col = jnp.bitwise_and(lane + d, D - 1)
                    tv = plsc.load_gather(t_rows, [rows_tc, col])
                    cv = plsc.load_gather(c_rows, [rows_tc, col])
                    out = [carry[0] + tv * cv]
                    for n in range(N):
                        nv = plsc.load_gather(n_rows, [rows_nb + n, col])
                        out.append(carry[1 + n] + nv * tv)
                    return tuple(out)

                init = tuple(jnp.zeros((L,), jnp.float32) for _ in range(N + 1))
                res = lax.fori_loop(0, D, body, init)

                p_flat = g * G + sg * L
                pos_v[p_flat // 128, pl.ds(p_flat % 128, L)] = res[0]
                for n in range(N):
                    n_flat = g * G * N + sg * L * N + n * L
                    neg_v[n_flat // 128, pl.ds(n_flat % 128, L)] = res[1 + n]

        pltpu.sync_copy(pos_v, pos_hbm.at[pl.ds(wid * PR, PR), :])
        pltpu.sync_copy(neg_v, neg_hbm.at[pl.ds(wid * NR, NR), :])

    return score_kernel(W_target, W_context, target_ids, context_ids, neg_ids)


def _tc_loss_body(pos_ref, neg_ref, out_ref):
    p = pos_ref[...]
    n = neg_ref[...]
    sp_pos = jnp.maximum(-p, 0.0) + jnp.log1p(jnp.exp(-jnp.abs(p)))
    sp_neg = jnp.maximum(n, 0.0) + jnp.log1p(jnp.exp(-jnp.abs(n)))
    out_ref[0, 0] = (jnp.sum(sp_pos) + jnp.sum(sp_neg)) / B


def _tc_loss(pos_s, neg_s):
    return pl.pallas_call(
        _tc_loss_body,
        out_shape=jax.ShapeDtypeStruct((1, 1), jnp.float32),
        out_specs=pl.BlockSpec(memory_space=pltpu.SMEM),
    )(pos_s, neg_s)[0, 0]


def kernel(W_target, W_context, target_ids, context_ids, neg_ids):
    neg_p = jnp.pad(neg_ids.astype(jnp.int32), ((0, 0), (0, 128 - N)))
    pos_s, neg_s = _sc_scores(W_target, W_context,
                              target_ids.astype(jnp.int32),
                              context_ids.astype(jnp.int32),
                              neg_p)
    return _tc_loss(pos_s, neg_s)


# speedup vs baseline: 1.0911x; 1.0911x over previous
import functools

import jax
import jax.numpy as jnp
from jax import lax
from jax.experimental import pallas as pl
from jax.experimental.pallas import tpu as pltpu
from jax.experimental.pallas import tpu_sc as plsc

V, D, B, N = 100000, 64, 16384, 20
NC, NS, L = 2, 16, 16
NW = NC * NS
BPW = B // NW
G = 32
NG = BPW // G
SG = G // L
IDX_CHUNK = 128
PR = BPW // 128
NR = BPW * N // 128


def _sc_scores(W_target, W_context, target_ids, context_ids, neg_ids):
    mesh = plsc.VectorSubcoreMesh(core_axis_name="c", subcore_axis_name="s")

    @functools.partial(
        pl.kernel,
        out_type=(
            jax.ShapeDtypeStruct((B // 128, 128), jnp.float32),
            jax.ShapeDtypeStruct((B * N // 128, 128), jnp.float32),
        ),
        mesh=mesh,
        scratch_types=[
            pltpu.VMEM((BPW,), jnp.int32),
            pltpu.VMEM((BPW,), jnp.int32),
            pltpu.VMEM((2, G, 128), jnp.int32),
            pltpu.VMEM((2, G * N), jnp.int32),
            pltpu.VMEM((2, G, D), jnp.float32),
            pltpu.VMEM((2, G, D), jnp.float32),
            pltpu.VMEM((2, G * N, D), jnp.float32),
            pltpu.VMEM((PR, 128), jnp.float32),
            pltpu.VMEM((NR, 128), jnp.float32),
            pltpu.SemaphoreType.DMA,
            pltpu.SemaphoreType.DMA,
        ],
        compiler_params=pltpu.CompilerParams(needs_layout_passes=False,
                                             use_tc_tiling_on_sc=False),
    )
    def score_kernel(wt_hbm, wc_hbm, tid_hbm, cid_hbm, nid_hbm,
                     pos_hbm, neg_hbm,
                     idx_t, idx_c, idx_n2, idx_n, t_rows, c_rows, n_rows,
                     pos_v, neg_v, sem0, sem1):
        wid = lax.axis_index("s") * NC + lax.axis_index("c")
        base = wid * BPW
        sems = [sem0, sem1]

        pltpu.sync_copy(tid_hbm.at[pl.ds(base, BPW)], idx_t)
        pltpu.sync_copy(cid_hbm.at[pl.ds(base, BPW)], idx_c)

        lane = lax.iota(jnp.int32, L)

        def launch(g):
            buf = g % 2
            pltpu.sync_copy(nid_hbm.at[pl.ds(base + g * G, G), :],
                            idx_n2.at[buf])

            def repack(k, _):
                for u in range(4):
                    j = (k * 4 + u) * L + lane
                    r = j // N
                    c = j - r * N
                    idx_n[buf, pl.ds((k * 4 + u) * L, L)] = plsc.load_gather(
                        idx_n2.at[buf], [r, c])
                return 0

            lax.fori_loop(0, G * N // (4 * L), repack, 0)

            copies = [
                pltpu.async_copy(wt_hbm.at[idx_t.at[pl.ds(g * G, G)]],
                                 t_rows.at[buf], sems[buf]),
                pltpu.async_copy(wc_hbm.at[idx_c.at[pl.ds(g * G, G)]],
                                 c_rows.at[buf], sems[buf]),
            ]
            for j in range(G * N // IDX_CHUNK):
                copies.append(pltpu.async_copy(
                    wc_hbm.at[idx_n.at[buf, pl.ds(j * IDX_CHUNK, IDX_CHUNK)]],
                    n_rows.at[buf].at[pl.ds(j * IDX_CHUNK, IDX_CHUNK)],
                    sems[buf]))
            return copies

        pending = launch(0)
        for g in range(NG):
            buf = g % 2
            for cp in pending:
                cp.wait()
            if g + 1 < NG:
                pending = launch(g + 1)

            for sg in range(SG):
                rows_tc = sg * L + lane
                rows_nb = rows_tc * N

                def body(d, carry, rows_tc=rows_tc, rows_nb=rows_nb, buf=buf):
                    col = jnp.bitwise_and(lane + d, D - 1)
                    tv = plsc.load_gather(t_rows.at[buf], [rows_tc, col])
                    cv = plsc.load_gather(c_rows.at[buf], [rows_tc, col])
                    out = [carry[0] + tv * cv]
                    for n in range(N):
                        nv = plsc.load_gather(n_rows.at[buf],
                                              [rows_nb + n, col])
                        out.append(carry[1 + n] + nv * tv)
                    return tuple(out)

                init = tuple(jnp.zeros((L,), jnp.float32) for _ in range(N + 1))
                res = lax.fori_loop(0, D, body, init)

                p_flat = g * G + sg * L
                pos_v[p_flat // 128, pl.ds(p_flat % 128, L)] = res[0]
                for n in range(N):
                    n_flat = g * G * N + sg * L * N + n * L
                    neg_v[n_flat // 128, pl.ds(n_flat % 128, L)] = res[1 + n]

        pltpu.sync_copy(pos_v, pos_hbm.at[pl.ds(wid * PR, PR), :])
        pltpu.sync_copy(neg_v, neg_hbm.at[pl.ds(wid * NR, NR), :])

    return score_kernel(W_target, W_context, target_ids, context_ids, neg_ids)


def _tc_loss_body(pos_ref, neg_ref, out_ref):
    p = pos_ref[...]
    n = neg_ref[...]
    sp_pos = jnp.maximum(-p, 0.0) + jnp.log1p(jnp.exp(-jnp.abs(p)))
    sp_neg = jnp.maximum(n, 0.0) + jnp.log1p(jnp.exp(-jnp.abs(n)))
    out_ref[0, 0] = (jnp.sum(sp_pos) + jnp.sum(sp_neg)) / B


def _tc_loss(pos_s, neg_s):
    return pl.pallas_call(
        _tc_loss_body,
        out_shape=jax.ShapeDtypeStruct((1, 1), jnp.float32),
        out_specs=pl.BlockSpec(memory_space=pltpu.SMEM),
    )(pos_s, neg_s)[0, 0]


def kernel(W_target, W_context, target_ids, context_ids, neg_ids):
    neg_p = jnp.pad(neg_ids.astype(jnp.int32), ((0, 0), (0, 128 - N)))
    pos_s, neg_s = _sc_scores(W_target, W_context,
                              target_ids.astype(jnp.int32),
                              context_ids.astype(jnp.int32),
                              neg_p)
    return _tc_loss(pos_s, neg_s)
